# baseline (speedup 1.0000x reference)
import functools

import jax
import jax.numpy as jnp
from jax import lax
from jax.experimental import pallas as pl
from jax.experimental.pallas import tpu as pltpu
from jax.experimental.pallas import tpu_sc as plsc

_VOCAB = 100000
_NUM_CLASSES = 2
_BATCH = 1024
_SEQ = 200
_N_SUBCORES = 16
_ROWS_PER = _BATCH // _N_SUBCORES
_LANES = 16
_GROUPS = _ROWS_PER // _LANES


def _bow_body(ids_hbm, w_hbm, out_hbm, table_v, ids_v, out_v, sem_w, sem_i):
    cls = lax.axis_index("c")
    chunk = lax.axis_index("s")
    rowbase = chunk * _ROWS_PER

    w_off = pl.multiple_of(cls * _VOCAB, 8)
    cp_w = pltpu.async_copy(w_hbm.at[pl.ds(w_off, _VOCAB)], table_v, sem_w)
    cp_i = pltpu.async_copy(ids_hbm.at[pl.ds(rowbase, _ROWS_PER), :], ids_v,
                            sem_i)
    cp_i.wait()
    cp_w.wait()

    lane = lax.iota(jnp.int32, _LANES)
    head = table_v[pl.ds(0, _LANES)]
    table_v[pl.ds(0, _LANES)] = jnp.where(lane == 0, jnp.float32(0.0), head)

    rows = [g * _LANES + lane for g in range(_GROUPS)]
    zero = jnp.zeros((_LANES,), jnp.float32)

    def step(t, accs):
        tvec = jnp.full((_LANES,), t, jnp.int32)
        ids16 = [plsc.load_gather(ids_v, [rows[g], tvec])
                 for g in range(_GROUPS)]
        vals = [plsc.load_gather(table_v, [ids16[g]]) for g in range(_GROUPS)]
        return tuple(accs[g] + vals[g] for g in range(_GROUPS))

    accs = lax.fori_loop(0, _SEQ, step, (zero,) * _GROUPS)
    for g in range(_GROUPS):
        out_v[pl.ds(g * _LANES, _LANES)] = accs[g]

    out_off = pl.multiple_of(cls * _BATCH + rowbase, 8)
    pltpu.sync_copy(out_v, out_hbm.at[pl.ds(out_off, _ROWS_PER)])


@jax.jit
def _bow_sc(ids, w_flat):
    mesh = plsc.VectorSubcoreMesh(core_axis_name="c", subcore_axis_name="s")
    f = functools.partial(
        pl.kernel,
        mesh=mesh,
        compiler_params=pltpu.CompilerParams(needs_layout_passes=False),
        out_type=jax.ShapeDtypeStruct((_NUM_CLASSES * _BATCH,), jnp.float32),
        scratch_types=[
            pltpu.VMEM((_VOCAB,), jnp.float32),
            pltpu.VMEM((_ROWS_PER, _SEQ), jnp.int32),
            pltpu.VMEM((_ROWS_PER,), jnp.float32),
            pltpu.SemaphoreType.DMA,
            pltpu.SemaphoreType.DMA,
        ],
    )(_bow_body)
    return f(ids, w_flat)


def kernel(input_ids, W, b):
    ids = input_ids.astype(jnp.int32)
    w_flat = W.astype(jnp.float32).reshape(-1)
    out = _bow_sc(ids, w_flat)
    return out.reshape(_NUM_CLASSES, _BATCH).T + b.astype(jnp.float32)
